# baseline (speedup 1.0000x reference)
import jax
import jax.numpy as jnp
from jax import lax
from jax.experimental import pallas as pl
from jax.experimental.pallas import tpu as pltpu
from jax.experimental.pallas import tpu_sc as plsc

NC, NS, L = 2, 16, 16
NW = NC * NS
B = 512
B3 = 256
Q3 = B3 // 128
Q = B // 128
RSUB = 2048

_SC_PARAMS = pltpu.CompilerParams(needs_layout_passes=False,
                                  use_tc_tiling_on_sc=False)

_BCAST_DN = lax.GatherDimensionNumbers(
    offset_dims=(), collapsed_slice_dims=(0,), start_index_map=(0,))


def _bcast(v16, r):
    idx = jnp.full((L, 1), r, jnp.int32)
    return lax.gather(v16, idx, _BCAST_DN, (1,),
                      mode=lax.GatherScatterMode.PROMISE_IN_BOUNDS)


def _atoms_body(nf_ref, w_ref, b_ref, out_ref):
    acc = jnp.dot(nf_ref[...], w_ref[...], preferred_element_type=jnp.float32)
    out_ref[...] = jax.nn.sigmoid(acc + b_ref[...])


def _compute_atoms(node_feat, W_atom, b_atom):
    n, _ = node_feat.shape
    nb = W_atom.shape[1]
    return pl.pallas_call(
        _atoms_body,
        out_shape=jax.ShapeDtypeStruct((n, nb), jnp.float32),
    )(node_feat, W_atom, b_atom.reshape(1, nb))


def _ve_body(atoms_hbm, gdst_hbm, cut_hbm, ve_hbm,
             gidx, arows, cutv, semL, semG, semO):
    cid = lax.axis_index("c")
    sid = lax.axis_index("s")
    wid = sid * NC + cid
    nbatch = cut_hbm.shape[0] // B
    lane = lax.iota(jnp.int32, L)
    n = (nbatch - wid + NW - 1) // NW

    def e0_of(j):
        return (wid + j * NW) * B

    def fire_linear(b, j):
        e0 = e0_of(j)
        pltpu.async_copy(gdst_hbm.at[pl.ds(e0 // 128, Q)], gidx.at[b], semL[b])
        pltpu.async_copy(cut_hbm.at[pl.ds(e0, B)], cutv.at[b], semL[b])

    def wait_linear(b):
        pltpu.make_async_copy(gdst_hbm.at[pl.ds(0, Q)], gidx.at[b], semL[b]).wait()
        pltpu.make_async_copy(cut_hbm.at[pl.ds(0, B)], cutv.at[b], semL[b]).wait()

    def fire_gather(b):
        for q in range(Q):
            pltpu.async_copy(atoms_hbm.at[gidx.at[b, q]],
                             arows.at[b, pl.ds(q * 128, 128)], semG[b])

    def wait_gather(b):
        for q in range(Q):
            pltpu.make_async_copy(atoms_hbm.at[gidx.at[b, q]],
                                  arows.at[b, pl.ds(q * 128, 128)], semG[b]).wait()

    def compute(b):
        halves = arows.shape[2] // L

        def gbody(g, _):
            c16 = cutv[b, pl.ds(g * L, L)]
            for r in range(L):
                cbc = _bcast(c16, r)
                row = g * L + r
                for h in range(halves):
                    sl = pl.ds(h * L, L)
                    arows[b, row, sl] = arows[b, row, sl] * cbc
            return 0

        lax.fori_loop(0, B // L, gbody, 0)

    def fire_out(b, j):
        pltpu.async_copy(arows.at[b], ve_hbm.at[pl.ds(e0_of(j), B)], semO[b])

    def wait_out(b):
        pltpu.make_async_copy(arows.at[b], ve_hbm.at[pl.ds(0, B)], semO[b]).wait()

    @pl.when(n > 0)
    def _():
        fire_linear(0, 0)

    @pl.when(n > 1)
    def _():
        fire_linear(1, 1)

    def pair(jj, _):
        j0 = 2 * jj
        j1 = j0 + 1
        for b, j in ((0, j0), (1, j1)):
            @pl.when(j < n)
            def _():
                wait_linear(b)

                @pl.when(j >= 2)
                def _():
                    wait_out(b)
                fire_gather(b)
                wait_gather(b)
                compute(b)
                fire_out(b, j)

                @pl.when(j + 2 < n)
                def _():
                    fire_linear(b, j + 2)
        return 0

    lax.fori_loop(0, (n + 1) // 2, pair, 0)

    @pl.when(n >= 1)
    def _():
        wait_out(0)

    @pl.when(n >= 2)
    def _():
        wait_out(1)


def _compute_ve(atoms, graph_dst, three_cutoff):
    e = graph_dst.shape[0]
    nb = atoms.shape[1]
    return pl.kernel(
        _ve_body,
        out_type=jax.ShapeDtypeStruct((e, nb), jnp.float32),
        mesh=plsc.VectorSubcoreMesh(core_axis_name="c", subcore_axis_name="s"),
        compiler_params=_SC_PARAMS,
        scratch_types=[
            pltpu.VMEM((2, Q, 128), jnp.int32),
            pltpu.VMEM((2, B, nb), jnp.float32),
            pltpu.VMEM((2, B), jnp.float32),
            [pltpu.SemaphoreType.DMA, pltpu.SemaphoreType.DMA],
            [pltpu.SemaphoreType.DMA, pltpu.SemaphoreType.DMA],
            [pltpu.SemaphoreType.DMA, pltpu.SemaphoreType.DMA],
        ],
    )(atoms, graph_dst.reshape(-1, 128), three_cutoff)


def _make_seg_body(nsub, nbasis):
    B, Q = B3, Q3
    nround = -(-nsub // NW)

    def _seg_body(ve_hbm, cut_hbm, basis_hbm, lgs_hbm, lgd_hbm, seg_hbm,
                  bnd_hbm, zeros_hbm, nb_hbm,
                  idxd, idxs, segv, basisv, rows, wv, bndv,
                  semL, semG, semZ, acc):
        cid = lax.axis_index("c")
        sid = lax.axis_index("s")
        wid = sid * NC + cid
        lane = lax.iota(jnp.int32, L)
        pltpu.sync_copy(bnd_hbm, bndv)

        def _scalar(j):
            v16 = bndv[pl.ds((j // L) * L, L)]
            return jnp.sum(jnp.where(lane == (j % L), v16, 0))

        for rnd in range(nround):
            sc = wid + rnd * NW
            base = sc * RSUB
            lo = (_scalar(sc) // B) * B
            hi = ((_scalar(sc + 1) + B - 1) // B) * B
            n = jnp.where(sc < nsub, (hi - lo) // B, 0)

            @pl.when(sc < nsub)
            def _():
                pltpu.async_copy(zeros_hbm, acc, semZ)
                pltpu.make_async_copy(zeros_hbm, acc, semZ).wait()

            def t0_of(j):
                return lo + j * B

            def fire_linear(b, j):
                t0 = t0_of(j)
                pltpu.async_copy(lgd_hbm.at[pl.ds(t0 // 128, Q)], idxd.at[b], semL[b])
                pltpu.async_copy(lgs_hbm.at[pl.ds(t0 // 128, Q)], idxs.at[b], semL[b])
                pltpu.async_copy(seg_hbm.at[pl.ds(t0, B)], segv.at[b], semL[b])
                pltpu.async_copy(basis_hbm.at[pl.ds(t0, B)], basisv.at[b], semL[b])

            def wait_linear(b):
                pltpu.make_async_copy(lgd_hbm.at[pl.ds(0, Q)], idxd.at[b], semL[b]).wait()
                pltpu.make_async_copy(lgs_hbm.at[pl.ds(0, Q)], idxs.at[b], semL[b]).wait()
                pltpu.make_async_copy(seg_hbm.at[pl.ds(0, B)], segv.at[b], semL[b]).wait()
                pltpu.make_async_copy(basis_hbm.at[pl.ds(0, B)], basisv.at[b], semL[b]).wait()

            def fire_gather(b):
                for q in range(Q):
                    pltpu.async_copy(ve_hbm.at[idxd.at[b, q]],
                                     rows.at[b, pl.ds(q * 128, 128)], semG[b])
                    pltpu.async_copy(cut_hbm.at[idxs.at[b, q]],
                                     wv.at[b, pl.ds(q * 128, 128)], semG[b])

            def wait_gather(b):
                for q in range(Q):
                    pltpu.make_async_copy(ve_hbm.at[idxd.at[b, q]],
                                          rows.at[b, pl.ds(q * 128, 128)], semG[b]).wait()
                    pltpu.make_async_copy(cut_hbm.at[idxs.at[b, q]],
                                          wv.at[b, pl.ds(q * 128, 128)], semG[b]).wait()

            def compute(b):
                halves = nbasis // L
                col = [lane + h * L for h in range(halves)]
                for q in range(Q):
                    def rbody(r8, _, q=q):
                        g = q * (128 // L) + r8
                        s16 = segv[b, pl.ds(g * L, L)]
                        rel = s16 - base
                        ok = (rel >= 0) & (rel < RSUB)
                        rel16 = jnp.where(ok, rel, RSUB)
                        w16 = wv[b, pl.ds(g * L, L)]
                        for r in range(L):
                            wbc = _bcast(w16, r)
                            rbc = _bcast(rel16, r)
                            row = g * L + r
                            for h in range(halves):
                                sl = pl.ds(h * L, L)
                                prod = (basisv[b, row, sl]
                                        * rows[b, row, sl] * wbc)
                                plsc.addupdate_scatter(acc, [rbc, col[h]], prod)
                        return 0

                    lax.fori_loop(0, 128 // L, rbody, 0)

            @pl.when(n > 0)
            def _():
                fire_linear(0, 0)

            @pl.when(n > 1)
            def _():
                fire_linear(1, 1)

            @pl.when(n > 0)
            def _():
                wait_linear(0)
                fire_gather(0)

            def pair(jj, _):
                j0 = 2 * jj
                j1 = j0 + 1
                @pl.when(j0 < n)
                def _():
                    wait_gather(0)

                @pl.when(j1 < n)
                def _():
                    wait_linear(1)
                    fire_gather(1)

                @pl.when(j0 < n)
                def _():
                    compute(0)

                @pl.when(j0 + 2 < n)
                def _():
                    fire_linear(0, j0 + 2)

                @pl.when(j1 < n)
                def _():
                    wait_gather(1)

                @pl.when(j0 + 2 < n)
                def _():
                    wait_linear(0)
                    fire_gather(0)

                @pl.when(j1 < n)
                def _():
                    compute(1)

                @pl.when(j1 + 2 < n)
                def _():
                    fire_linear(1, j1 + 2)
                return 0

            lax.fori_loop(0, (n + 1) // 2, pair, 0)

            @pl.when(sc < nsub)
            def _():
                pltpu.sync_copy(acc.at[pl.ds(0, RSUB)],
                                nb_hbm.at[pl.ds(base, RSUB)])

    return _seg_body


def _segment_accumulate(ve, three_cutoff, three_basis, lg_src, lg_dst,
                        segment_ids, nsub):
    B, Q = B3, Q3
    nb = ve.shape[1]
    nbnd = -(-(nsub + 1) // L) * L
    bounds = jnp.searchsorted(
        segment_ids,
        jnp.arange(nsub + 1, dtype=jnp.int32) * RSUB).astype(jnp.int32)
    bnd = jnp.zeros((nbnd,), jnp.int32).at[:nsub + 1].set(bounds)
    zeros = jnp.zeros((RSUB + 8, nb), jnp.float32)
    dma2 = [pltpu.SemaphoreType.DMA, pltpu.SemaphoreType.DMA]
    return pl.kernel(
        _make_seg_body(nsub, nb),
        out_type=jax.ShapeDtypeStruct((nsub * RSUB, nb), jnp.float32),
        mesh=plsc.VectorSubcoreMesh(core_axis_name="c", subcore_axis_name="s"),
        compiler_params=_SC_PARAMS,
        scratch_types=[
            pltpu.VMEM((2, Q, 128), jnp.int32),
            pltpu.VMEM((2, Q, 128), jnp.int32),
            pltpu.VMEM((2, B), jnp.int32),
            pltpu.VMEM((2, B, nb), jnp.float32),
            pltpu.VMEM((2, B, nb), jnp.float32),
            pltpu.VMEM((2, B), jnp.float32),
            pltpu.VMEM((nbnd,), jnp.int32),
            dma2,
            dma2,
            pltpu.SemaphoreType.DMA,
            pltpu.VMEM((RSUB + 8, nb), jnp.float32),
        ],
    )(ve, three_cutoff, three_basis, lg_src.reshape(-1, 128),
      lg_dst.reshape(-1, 128), segment_ids, bnd, zeros)


BLK3 = 2560


def _mlp_body(nb_ref, ef_ref, wo_ref, bo_ref, wg_ref, bg_ref, out_ref):
    x = nb_ref[...]
    h = jnp.dot(x, wo_ref[...], preferred_element_type=jnp.float32) + bo_ref[...]
    g = jnp.dot(x, wg_ref[...], preferred_element_type=jnp.float32) + bg_ref[...]
    out_ref[...] = ef_ref[...] + jax.nn.silu(h) * jax.nn.sigmoid(g)


def _gated_mlp(nbond, edge_feat, W_out, b_out, W_gate, b_gate):
    e, d = edge_feat.shape
    nb = nbond.shape[1]
    return pl.pallas_call(
        _mlp_body,
        grid=(e // BLK3,),
        in_specs=[
            pl.BlockSpec((BLK3, nb), lambda i: (i, 0)),
            pl.BlockSpec((BLK3, d), lambda i: (i, 0)),
            pl.BlockSpec((nb, d), lambda i: (0, 0)),
            pl.BlockSpec((1, d), lambda i: (0, 0)),
            pl.BlockSpec((nb, d), lambda i: (0, 0)),
            pl.BlockSpec((1, d), lambda i: (0, 0)),
        ],
        out_specs=pl.BlockSpec((BLK3, d), lambda i: (i, 0)),
        out_shape=jax.ShapeDtypeStruct((e, d), jnp.float32),
    )(nbond, edge_feat, W_out, b_out.reshape(1, d), W_gate, b_gate.reshape(1, d))


def kernel(node_feat, edge_feat, three_basis, three_cutoff, graph_dst,
           lg_src, lg_dst, segment_ids, W_atom, b_atom, W_out, b_out,
           W_gate, b_gate):
    e = edge_feat.shape[0]
    t = three_basis.shape[0]
    assert e % B == 0 and t % B == 0
    nsub = -(-e // RSUB)

    atoms = _compute_atoms(node_feat, W_atom, b_atom)
    ve = _compute_ve(atoms, graph_dst, three_cutoff)
    nb_pad = _segment_accumulate(ve, three_cutoff, three_basis, lg_src,
                                 lg_dst, segment_ids, nsub)
    return _gated_mlp(nb_pad[:e], edge_feat, W_out, b_out, W_gate, b_gate)


# speedup vs baseline: 1.1074x; 1.1074x over previous
import jax
import jax.numpy as jnp
from jax import lax
from jax.experimental import pallas as pl
from jax.experimental.pallas import tpu as pltpu
from jax.experimental.pallas import tpu_sc as plsc

NC, NS, L = 2, 16, 16
NW = NC * NS
B = 512
B3 = 256
Q3 = B3 // 128
Q = B // 128
RSUB = 2048

_SC_PARAMS = pltpu.CompilerParams(needs_layout_passes=False,
                                  use_tc_tiling_on_sc=False)

_BCAST_DN = lax.GatherDimensionNumbers(
    offset_dims=(), collapsed_slice_dims=(0,), start_index_map=(0,))


def _bcast(v16, r):
    idx = jnp.full((L, 1), r, jnp.int32)
    return lax.gather(v16, idx, _BCAST_DN, (1,),
                      mode=lax.GatherScatterMode.PROMISE_IN_BOUNDS)


def _atoms_body(nf_ref, w_ref, b_ref, out_ref):
    acc = jnp.dot(nf_ref[...], w_ref[...], preferred_element_type=jnp.float32)
    out_ref[...] = jax.nn.sigmoid(acc + b_ref[...])


def _compute_atoms(node_feat, W_atom, b_atom):
    n, _ = node_feat.shape
    nb = W_atom.shape[1]
    return pl.pallas_call(
        _atoms_body,
        out_shape=jax.ShapeDtypeStruct((n, nb), jnp.float32),
    )(node_feat, W_atom, b_atom.reshape(1, nb))


def _ve_body(atoms_hbm, gdst_hbm, cut_hbm, ve_hbm,
             gidx, arows, packv, cutv, semL, semG, semO):
    cid = lax.axis_index("c")
    sid = lax.axis_index("s")
    wid = sid * NC + cid
    nbatch = cut_hbm.shape[0] // B
    lane = lax.iota(jnp.int32, L)
    n = (nbatch - wid + NW - 1) // NW

    def e0_of(j):
        return (wid + j * NW) * B

    def fire_linear(b, j):
        e0 = e0_of(j)
        pltpu.async_copy(gdst_hbm.at[pl.ds(e0 // 128, Q)], gidx.at[b], semL[b])
        pltpu.async_copy(cut_hbm.at[pl.ds(e0, B)], cutv.at[b], semL[b])

    def wait_linear(b):
        pltpu.make_async_copy(gdst_hbm.at[pl.ds(0, Q)], gidx.at[b], semL[b]).wait()
        pltpu.make_async_copy(cut_hbm.at[pl.ds(0, B)], cutv.at[b], semL[b]).wait()

    def fire_gather(b):
        for q in range(Q):
            pltpu.async_copy(atoms_hbm.at[gidx.at[b, q]],
                             arows.at[b, pl.ds(q * 128, 128)], semG[b])

    def wait_gather(b):
        for q in range(Q):
            pltpu.make_async_copy(atoms_hbm.at[gidx.at[b, q]],
                                  arows.at[b, pl.ds(q * 128, 128)], semG[b]).wait()

    def compute(b):
        def gbody(g, _):
            c16 = cutv[b, pl.ds(g * L, L)]
            for r in range(L):
                cbc = _bcast(c16, r)
                row = g * L + r
                h0 = arows[b, row, pl.ds(0, L)] * cbc
                h1 = arows[b, row, pl.ds(L, L)] * cbc
                packv[b, row, :] = plsc.pack(
                    h0, h1, format=plsc.PackFormat.INTERLEAVED)
            return 0

        lax.fori_loop(0, B // L, gbody, 0)

    def fire_out(b, j):
        pltpu.async_copy(packv.at[b], ve_hbm.at[pl.ds(e0_of(j), B)], semO[b])

    def wait_out(b):
        pltpu.make_async_copy(packv.at[b], ve_hbm.at[pl.ds(0, B)], semO[b]).wait()

    @pl.when(n > 0)
    def _():
        fire_linear(0, 0)

    @pl.when(n > 1)
    def _():
        fire_linear(1, 1)

    def pair(jj, _):
        j0 = 2 * jj
        j1 = j0 + 1
        for b, j in ((0, j0), (1, j1)):
            @pl.when(j < n)
            def _():
                wait_linear(b)

                @pl.when(j >= 2)
                def _():
                    wait_out(b)
                fire_gather(b)
                wait_gather(b)
                compute(b)
                fire_out(b, j)

                @pl.when(j + 2 < n)
                def _():
                    fire_linear(b, j + 2)
        return 0

    lax.fori_loop(0, (n + 1) // 2, pair, 0)

    @pl.when(n >= 1)
    def _():
        wait_out(0)

    @pl.when(n >= 2)
    def _():
        wait_out(1)


def _compute_ve(atoms, graph_dst, three_cutoff):
    e = graph_dst.shape[0]
    nb = atoms.shape[1]
    return pl.kernel(
        _ve_body,
        out_type=jax.ShapeDtypeStruct((e, nb), jnp.bfloat16),
        mesh=plsc.VectorSubcoreMesh(core_axis_name="c", subcore_axis_name="s"),
        compiler_params=_SC_PARAMS,
        scratch_types=[
            pltpu.VMEM((2, Q, 128), jnp.int32),
            pltpu.VMEM((2, B, nb), jnp.float32),
            pltpu.VMEM((2, B, nb), jnp.bfloat16),
            pltpu.VMEM((2, B), jnp.float32),
            [pltpu.SemaphoreType.DMA, pltpu.SemaphoreType.DMA],
            [pltpu.SemaphoreType.DMA, pltpu.SemaphoreType.DMA],
            [pltpu.SemaphoreType.DMA, pltpu.SemaphoreType.DMA],
        ],
    )(atoms, graph_dst.reshape(-1, 128), three_cutoff)


def _make_seg_body(nsub, nbasis):
    B, Q = B3, Q3
    nround = -(-nsub // NW)

    def _seg_body(ve_hbm, cut_hbm, basis_hbm, lgs_hbm, lgd_hbm, seg_hbm,
                  bnd_hbm, zeros_hbm, nb_hbm,
                  idxd, idxs, segv, basisv, rows, wv, bndv,
                  semL, semG, semZ, acc):
        cid = lax.axis_index("c")
        sid = lax.axis_index("s")
        wid = sid * NC + cid
        lane = lax.iota(jnp.int32, L)
        pltpu.sync_copy(bnd_hbm, bndv)

        def _scalar(j):
            v16 = bndv[pl.ds((j // L) * L, L)]
            return jnp.sum(jnp.where(lane == (j % L), v16, 0))

        for rnd in range(nround):
            sc = wid + rnd * NW
            base = sc * RSUB
            lo = (_scalar(sc) // B) * B
            hi = ((_scalar(sc + 1) + B - 1) // B) * B
            n = jnp.where(sc < nsub, (hi - lo) // B, 0)

            @pl.when(sc < nsub)
            def _():
                pltpu.async_copy(zeros_hbm, acc, semZ)
                pltpu.make_async_copy(zeros_hbm, acc, semZ).wait()

            def t0_of(j):
                return lo + j * B

            def fire_linear(b, j):
                t0 = t0_of(j)
                pltpu.async_copy(lgd_hbm.at[pl.ds(t0 // 128, Q)], idxd.at[b], semL[b])
                pltpu.async_copy(lgs_hbm.at[pl.ds(t0 // 128, Q)], idxs.at[b], semL[b])
                pltpu.async_copy(seg_hbm.at[pl.ds(t0, B)], segv.at[b], semL[b])
                pltpu.async_copy(basis_hbm.at[pl.ds(t0, B)], basisv.at[b], semL[b])

            def wait_linear(b):
                pltpu.make_async_copy(lgd_hbm.at[pl.ds(0, Q)], idxd.at[b], semL[b]).wait()
                pltpu.make_async_copy(lgs_hbm.at[pl.ds(0, Q)], idxs.at[b], semL[b]).wait()
                pltpu.make_async_copy(seg_hbm.at[pl.ds(0, B)], segv.at[b], semL[b]).wait()
                pltpu.make_async_copy(basis_hbm.at[pl.ds(0, B)], basisv.at[b], semL[b]).wait()

            def fire_gather(b):
                for q in range(Q):
                    pltpu.async_copy(ve_hbm.at[idxd.at[b, q]],
                                     rows.at[b, pl.ds(q * 128, 128)], semG[b])
                    pltpu.async_copy(cut_hbm.at[idxs.at[b, q]],
                                     wv.at[b, pl.ds(q * 128, 128)], semG[b])

            def wait_gather(b):
                for q in range(Q):
                    pltpu.make_async_copy(ve_hbm.at[idxd.at[b, q]],
                                          rows.at[b, pl.ds(q * 128, 128)], semG[b]).wait()
                    pltpu.make_async_copy(cut_hbm.at[idxs.at[b, q]],
                                          wv.at[b, pl.ds(q * 128, 128)], semG[b]).wait()

            def compute(b):
                halves = nbasis // L
                col = [lane + h * L for h in range(halves)]
                for q in range(Q):
                    def rbody(r8, _, q=q):
                        g = q * (128 // L) + r8
                        s16 = segv[b, pl.ds(g * L, L)]
                        rel = s16 - base
                        ok = (rel >= 0) & (rel < RSUB)
                        rel16 = jnp.where(ok, rel, RSUB)
                        w16 = wv[b, pl.ds(g * L, L)]
                        for r in range(L):
                            wbc = _bcast(w16, r)
                            rbc = _bcast(rel16, r)
                            row = g * L + r
                            vh = plsc.unpack(rows[b, row, :],
                                             format=plsc.PackFormat.INTERLEAVED)
                            for h in range(halves):
                                sl = pl.ds(h * L, L)
                                prod = basisv[b, row, sl] * vh[h] * wbc
                                plsc.addupdate_scatter(acc, [rbc, col[h]], prod)
                        return 0

                    lax.fori_loop(0, 128 // L, rbody, 0)

            @pl.when(n > 0)
            def _():
                fire_linear(0, 0)

            @pl.when(n > 1)
            def _():
                fire_linear(1, 1)

            @pl.when(n > 0)
            def _():
                wait_linear(0)
                fire_gather(0)

            def pair(jj, _):
                j0 = 2 * jj
                j1 = j0 + 1
                @pl.when(j0 < n)
                def _():
                    wait_gather(0)

                @pl.when(j1 < n)
                def _():
                    wait_linear(1)
                    fire_gather(1)

                @pl.when(j0 < n)
                def _():
                    compute(0)

                @pl.when(j0 + 2 < n)
                def _():
                    fire_linear(0, j0 + 2)

                @pl.when(j1 < n)
                def _():
                    wait_gather(1)

                @pl.when(j0 + 2 < n)
                def _():
                    wait_linear(0)
                    fire_gather(0)

                @pl.when(j1 < n)
                def _():
                    compute(1)

                @pl.when(j1 + 2 < n)
                def _():
                    fire_linear(1, j1 + 2)
                return 0

            lax.fori_loop(0, (n + 1) // 2, pair, 0)

            @pl.when(sc < nsub)
            def _():
                pltpu.sync_copy(acc.at[pl.ds(0, RSUB)],
                                nb_hbm.at[pl.ds(base, RSUB)])

    return _seg_body


def _segment_accumulate(ve, three_cutoff, three_basis, lg_src, lg_dst,
                        segment_ids, nsub):
    B, Q = B3, Q3
    nb = ve.shape[1]
    nbnd = -(-(nsub + 1) // L) * L
    bounds = jnp.searchsorted(
        segment_ids,
        jnp.arange(nsub + 1, dtype=jnp.int32) * RSUB).astype(jnp.int32)
    bnd = jnp.zeros((nbnd,), jnp.int32).at[:nsub + 1].set(bounds)
    zeros = jnp.zeros((RSUB + 8, nb), jnp.float32)
    dma2 = [pltpu.SemaphoreType.DMA, pltpu.SemaphoreType.DMA]
    return pl.kernel(
        _make_seg_body(nsub, nb),
        out_type=jax.ShapeDtypeStruct((nsub * RSUB, nb), jnp.float32),
        mesh=plsc.VectorSubcoreMesh(core_axis_name="c", subcore_axis_name="s"),
        compiler_params=_SC_PARAMS,
        scratch_types=[
            pltpu.VMEM((2, Q, 128), jnp.int32),
            pltpu.VMEM((2, Q, 128), jnp.int32),
            pltpu.VMEM((2, B), jnp.int32),
            pltpu.VMEM((2, B, nb), jnp.float32),
            pltpu.VMEM((2, B, nb), jnp.bfloat16),
            pltpu.VMEM((2, B), jnp.float32),
            pltpu.VMEM((nbnd,), jnp.int32),
            dma2,
            dma2,
            pltpu.SemaphoreType.DMA,
            pltpu.VMEM((RSUB + 8, nb), jnp.float32),
        ],
    )(ve, three_cutoff, three_basis, lg_src.reshape(-1, 128),
      lg_dst.reshape(-1, 128), segment_ids, bnd, zeros)


BLK3 = 6400


def _mlp_body(nb_ref, ef_ref, wo_ref, bo_ref, wg_ref, bg_ref, out_ref):
    x = nb_ref[...]
    h = jnp.dot(x, wo_ref[...], preferred_element_type=jnp.float32) + bo_ref[...]
    g = jnp.dot(x, wg_ref[...], preferred_element_type=jnp.float32) + bg_ref[...]
    out_ref[...] = ef_ref[...] + jax.nn.silu(h) * jax.nn.sigmoid(g)


def _gated_mlp(nbond, edge_feat, W_out, b_out, W_gate, b_gate):
    e, d = edge_feat.shape
    nb = nbond.shape[1]
    return pl.pallas_call(
        _mlp_body,
        grid=(e // BLK3,),
        in_specs=[
            pl.BlockSpec((BLK3, nb), lambda i: (i, 0)),
            pl.BlockSpec((BLK3, d), lambda i: (i, 0)),
            pl.BlockSpec((nb, d), lambda i: (0, 0)),
            pl.BlockSpec((1, d), lambda i: (0, 0)),
            pl.BlockSpec((nb, d), lambda i: (0, 0)),
            pl.BlockSpec((1, d), lambda i: (0, 0)),
        ],
        out_specs=pl.BlockSpec((BLK3, d), lambda i: (i, 0)),
        out_shape=jax.ShapeDtypeStruct((e, d), jnp.float32),
    )(nbond, edge_feat, W_out, b_out.reshape(1, d), W_gate, b_gate.reshape(1, d))


def kernel(node_feat, edge_feat, three_basis, three_cutoff, graph_dst,
           lg_src, lg_dst, segment_ids, W_atom, b_atom, W_out, b_out,
           W_gate, b_gate):
    e = edge_feat.shape[0]
    t = three_basis.shape[0]
    assert e % B == 0 and t % B == 0
    nsub = -(-e // RSUB)

    atoms = _compute_atoms(node_feat, W_atom, b_atom)
    ve = _compute_ve(atoms, graph_dst, three_cutoff)
    nb_pad = _segment_accumulate(ve, three_cutoff, three_basis, lg_src,
                                 lg_dst, segment_ids, nsub)
    return _gated_mlp(nb_pad, edge_feat, W_out, b_out, W_gate, b_gate)
